# baseline (speedup 1.0000x reference)
import functools

import jax
import jax.numpy as jnp
from jax import lax
from jax.experimental import pallas as pl
from jax.experimental.pallas import tpu as pltpu
from jax.experimental.pallas import tpu_sc as plsc

BATCH = 16384
DIM = 512
NUM_CORES = 2
NUM_SUBCORES = 16
NUM_WORKERS = NUM_CORES * NUM_SUBCORES
ROWS_PER_WORKER = BATCH // NUM_WORKERS
CHUNK = 32
NUM_CHUNKS = ROWS_PER_WORKER // CHUNK
NBUF = 7

_mesh = plsc.VectorSubcoreMesh(core_axis_name="c", subcore_axis_name="s")


@functools.partial(
    pl.kernel,
    mesh=_mesh,
    out_type=jax.ShapeDtypeStruct((BATCH, DIM), jnp.float32),
    scratch_types=[
        pltpu.VMEM((ROWS_PER_WORKER,), jnp.int32),
        pltpu.VMEM((NBUF, CHUNK, DIM), jnp.float32),
    ] + [pltpu.SemaphoreType.DMA] * (2 * NBUF),
)
def _sc_gather(idx_hbm, table_hbm, out_hbm, idx_v, rows_v, *sems):
    gsems, ssems = sems[:NBUF], sems[NBUF:]
    wid = lax.axis_index("s") * NUM_CORES + lax.axis_index("c")
    base = wid * ROWS_PER_WORKER
    pltpu.sync_copy(idx_hbm.at[pl.ds(base, CHUNK)],
                    idx_v.at[pl.ds(0, CHUNK)])

    gathers = [None] * NUM_CHUNKS
    stores = [None] * NUM_CHUNKS
    gathers[0] = pltpu.async_copy(
        table_hbm.at[idx_v.at[pl.ds(0, CHUNK)]], rows_v.at[0], gsems[0])
    pltpu.sync_copy(
        idx_hbm.at[pl.ds(base + CHUNK, ROWS_PER_WORKER - CHUNK)],
        idx_v.at[pl.ds(CHUNK, ROWS_PER_WORKER - CHUNK)])
    for j in range(1, min(NBUF - 1, NUM_CHUNKS)):
        gathers[j] = pltpu.async_copy(
            table_hbm.at[idx_v.at[pl.ds(j * CHUNK, CHUNK)]],
            rows_v.at[j % NBUF], gsems[j % NBUF])
    for j in range(NUM_CHUNKS):
        b = j % NBUF
        gathers[j].wait()
        stores[j] = pltpu.async_copy(
            rows_v.at[b], out_hbm.at[pl.ds(base + j * CHUNK, CHUNK)],
            ssems[b])
        nj = j + NBUF - 1
        if nj < NUM_CHUNKS:
            if nj - NBUF >= 0:
                stores[nj - NBUF].wait()
            gathers[nj] = pltpu.async_copy(
                table_hbm.at[idx_v.at[pl.ds(nj * CHUNK, CHUNK)]],
                rows_v.at[nj % NBUF], gsems[nj % NBUF])
    for j in range(max(0, NUM_CHUNKS - NBUF), NUM_CHUNKS):
        stores[j].wait()


def kernel(timesteps, pos_encoding):
    return _sc_gather(timesteps.astype(jnp.int32), pos_encoding)


# speedup vs baseline: 1.0252x; 1.0252x over previous
import functools

import jax
import jax.numpy as jnp
from jax import lax
from jax.experimental import pallas as pl
from jax.experimental.pallas import tpu as pltpu
from jax.experimental.pallas import tpu_sc as plsc

BATCH = 16384
DIM = 512
NUM_CORES = 2
NUM_SUBCORES = 16
NUM_WORKERS = NUM_CORES * NUM_SUBCORES
ROWS_PER_WORKER = BATCH // NUM_WORKERS
CHUNK = 32
NUM_CHUNKS = ROWS_PER_WORKER // CHUNK
NBUF = 7

_mesh = plsc.VectorSubcoreMesh(core_axis_name="c", subcore_axis_name="s")


@functools.partial(
    pl.kernel,
    mesh=_mesh,
    out_type=jax.ShapeDtypeStruct((BATCH, DIM), jnp.float32),
    scratch_types=[
        pltpu.VMEM((ROWS_PER_WORKER,), jnp.int32),
        pltpu.VMEM((NBUF, CHUNK, DIM), jnp.float32),
    ] + [pltpu.SemaphoreType.DMA] * (2 * NBUF),
)
def _sc_gather(idx_hbm, table_hbm, out_hbm, idx_v, rows_v, *sems):
    gsems, ssems = sems[:NBUF], sems[NBUF:]
    wid = lax.axis_index("s") * NUM_CORES + lax.axis_index("c")
    base = wid * ROWS_PER_WORKER
    pltpu.sync_copy(idx_hbm.at[pl.ds(base, ROWS_PER_WORKER)], idx_v)

    gathers = [None] * NUM_CHUNKS
    stores = [None] * NUM_CHUNKS
    for j in range(min(NBUF - 1, NUM_CHUNKS)):
        gathers[j] = pltpu.async_copy(
            table_hbm.at[idx_v.at[pl.ds(j * CHUNK, CHUNK)]],
            rows_v.at[j % NBUF], gsems[j % NBUF])
    for j in range(NUM_CHUNKS):
        b = j % NBUF
        gathers[j].wait()
        stores[j] = pltpu.async_copy(
            rows_v.at[b], out_hbm.at[pl.ds(base + j * CHUNK, CHUNK)],
            ssems[b])
        nj = j + NBUF - 1
        if nj < NUM_CHUNKS:
            if nj - NBUF >= 0:
                stores[nj - NBUF].wait()
            gathers[nj] = pltpu.async_copy(
                table_hbm.at[idx_v.at[pl.ds(nj * CHUNK, CHUNK)]],
                rows_v.at[nj % NBUF], gsems[nj % NBUF])
    for j in range(max(0, NUM_CHUNKS - NBUF), NUM_CHUNKS):
        stores[j].wait()


def kernel(timesteps, pos_encoding):
    return _sc_gather(timesteps.astype(jnp.int32), pos_encoding)
